# baseline (speedup 1.0000x reference)
import functools

import jax
import jax.numpy as jnp
from jax.experimental import pallas as pl
from jax.experimental.pallas import tpu as pltpu

B, N, D, H, R = 4096, 200, 16, 16, 8
BB = 64


def _leaky(x):
    return jnp.where(x >= 0, x, 0.01 * x)


def _block_kernel(u_ref, i_ref, a_ref, o_ref, s_ref,
                  wui1_ref, bui1_ref, wui2_ref, bui2_ref,
                  w1_ref, b1_ref, w2_ref, b2_ref,
                  out_ref):
    ui_in = jnp.concatenate([u_ref[...], i_ref[...]], axis=-1)
    h_ui = _leaky(jnp.dot(ui_in, wui1_ref[...],
                          preferred_element_type=jnp.float32) + bui1_ref[...])
    ui_emb = _leaky(jnp.dot(h_ui, wui2_ref[...],
                            preferred_element_type=jnp.float32) + bui2_ref[...])

    x = jnp.concatenate([a_ref[...], o_ref[...]], axis=-1)
    x2 = x.reshape(BB * N, 2 * D)
    h_all = _leaky(jnp.dot(x2, w1_ref[...],
                           preferred_element_type=jnp.float32) + b1_ref[...])
    out_all = _leaky(jnp.dot(h_all, w2_ref[...],
                             preferred_element_type=jnp.float32) + b2_ref[...])
    out3 = out_all.reshape(BB, N, R * H)

    lane_expert = jax.lax.broadcasted_iota(jnp.int32, (BB, N, R * H), 2) // H
    sel = lane_expert == s_ref[...][:, :, None]
    ui_tiled = jnp.concatenate([ui_emb] * R, axis=-1)
    um = jnp.where(sel, ui_tiled[:, None, :], 0.0)
    out_ref[...] = jnp.sum(out3 * um, axis=-1)


@jax.jit
def _run(u_emb, i_emb, a_emb, o_emb, s,
         Wui1, bui1, Wui2, bui2, w1_all, b1_all, w2_bd, b2_all):
    grid = (B // BB,)
    full = lambda shape: pl.BlockSpec(shape, lambda b: (0,) * len(shape))
    return pl.pallas_call(
        _block_kernel,
        grid=grid,
        in_specs=[
            pl.BlockSpec((BB, D), lambda b: (b, 0)),
            pl.BlockSpec((BB, D), lambda b: (b, 0)),
            pl.BlockSpec((BB, N, D), lambda b: (b, 0, 0)),
            pl.BlockSpec((BB, N, D), lambda b: (b, 0, 0)),
            pl.BlockSpec((BB, N), lambda b: (b, 0)),
            full((2 * D, H)), full((H,)), full((H, H)), full((H,)),
            full((2 * D, R * H)), full((R * H,)),
            full((R * H, R * H)), full((R * H,)),
        ],
        out_specs=pl.BlockSpec((BB, N), lambda b: (b, 0)),
        out_shape=jax.ShapeDtypeStruct((B, N), jnp.float32),
        compiler_params=pltpu.CompilerParams(
            dimension_semantics=("arbitrary",),
        ),
    )(u_emb, i_emb, a_emb, o_emb, s,
      Wui1, bui1, Wui2, bui2, w1_all, b1_all, w2_bd, b2_all)


def kernel(u_emb, i_emb, a_emb, o_emb, s,
           Wui1, bui1, Wui2, bui2, Wao1, bao1, Wao2, bao2):
    w1_all = jnp.transpose(Wao1, (1, 0, 2)).reshape(2 * D, R * H)
    b1_all = bao1.reshape(R * H)
    eye = jnp.eye(R, dtype=Wao2.dtype)
    w2_bd = jnp.einsum('rkj,rq->rkqj', Wao2, eye).reshape(R * H, R * H)
    b2_all = bao2.reshape(R * H)
    return _run(u_emb, i_emb, a_emb, o_emb, s,
                Wui1, bui1, Wui2, bui2, w1_all, b1_all, w2_bd, b2_all)


# speedup vs baseline: 1.0250x; 1.0250x over previous
import jax
import jax.numpy as jnp
from jax.experimental import pallas as pl
from jax.experimental.pallas import tpu as pltpu

B, N, D, H, R = 4096, 200, 16, 16, 8
BB = 64


def _leaky(x):
    return jnp.maximum(x, 0.01 * x)


def _block_kernel(u_ref, i_ref, a_ref, o_ref, s_ref,
                  wui1a_ref, wui1b_ref, bui1_ref, wui2_ref, bui2_ref,
                  w1a_ref, w1b_ref, b1_ref, w2_ref, b2_ref, g_ref,
                  out_ref):
    f32 = jnp.float32
    h_ui = _leaky(jnp.dot(u_ref[...], wui1a_ref[...], preferred_element_type=f32)
                  + jnp.dot(i_ref[...], wui1b_ref[...], preferred_element_type=f32)
                  + bui1_ref[...])
    ui_emb = _leaky(jnp.dot(h_ui, wui2_ref[...], preferred_element_type=f32)
                    + bui2_ref[...])
    ui_t = jnp.concatenate([ui_emb] * R, axis=-1)

    a2 = a_ref[...].reshape(BB * N, D)
    o2 = o_ref[...].reshape(BB * N, D)
    h_all = _leaky(jnp.dot(a2, w1a_ref[...], preferred_element_type=f32)
                   + jnp.dot(o2, w1b_ref[...], preferred_element_type=f32)
                   + b1_ref[...])
    out_all = _leaky(jnp.dot(h_all, w2_ref[...], preferred_element_type=f32)
                     + b2_ref[...])

    ou = out_all.reshape(BB, N, R * H) * ui_t[:, None, :]
    scores = jnp.dot(ou.reshape(BB * N, R * H), g_ref[...],
                     preferred_element_type=f32).reshape(BB, N, R)
    oh = jax.lax.broadcasted_iota(jnp.int32, (BB, N, R), 2) == s_ref[...][:, :, None]
    out_ref[...] = jnp.sum(jnp.where(oh, scores, 0.0), axis=-1)


@jax.jit
def _run(u_emb, i_emb, a_emb, o_emb, s,
         wui1a, wui1b, bui1, Wui2, bui2, w1a, w1b, b1_all, w2_bd, b2_all, g):
    grid = (B // BB,)
    full = lambda shape: pl.BlockSpec(shape, lambda b: (0,) * len(shape))
    return pl.pallas_call(
        _block_kernel,
        grid=grid,
        in_specs=[
            pl.BlockSpec((BB, D), lambda b: (b, 0)),
            pl.BlockSpec((BB, D), lambda b: (b, 0)),
            pl.BlockSpec((BB, N, D), lambda b: (b, 0, 0)),
            pl.BlockSpec((BB, N, D), lambda b: (b, 0, 0)),
            pl.BlockSpec((BB, N), lambda b: (b, 0)),
            full((D, H)), full((D, H)), full((H,)), full((H, H)), full((H,)),
            full((D, R * H)), full((D, R * H)), full((R * H,)),
            full((R * H, R * H)), full((R * H,)), full((R * H, R)),
        ],
        out_specs=pl.BlockSpec((BB, N), lambda b: (b, 0)),
        out_shape=jax.ShapeDtypeStruct((B, N), jnp.float32),
        compiler_params=pltpu.CompilerParams(
            dimension_semantics=("arbitrary",),
        ),
    )(u_emb, i_emb, a_emb, o_emb, s,
      wui1a, wui1b, bui1, Wui2, bui2, w1a, w1b, b1_all, w2_bd, b2_all, g)


def kernel(u_emb, i_emb, a_emb, o_emb, s,
           Wui1, bui1, Wui2, bui2, Wao1, bao1, Wao2, bao2):
    w1_all = jnp.transpose(Wao1, (1, 0, 2)).reshape(2 * D, R * H)
    b1_all = bao1.reshape(R * H)
    eye = jnp.eye(R, dtype=Wao2.dtype)
    w2_bd = jnp.einsum('rkj,rq->rkqj', Wao2, eye).reshape(R * H, R * H)
    b2_all = bao2.reshape(R * H)
    g = jnp.repeat(jnp.eye(R, dtype=jnp.float32), H, axis=0)
    return _run(u_emb, i_emb, a_emb, o_emb, s,
                Wui1[:D], Wui1[D:], bui1, Wui2, bui2,
                w1_all[:D], w1_all[D:], b1_all, w2_bd, b2_all, g)


# speedup vs baseline: 1.6062x; 1.5670x over previous
import jax
import jax.numpy as jnp
from jax.experimental import pallas as pl
from jax.experimental.pallas import tpu as pltpu

B, N, D, H, R = 4096, 200, 16, 16, 8
BBLK = 32
TB = BBLK * N


def _leaky(x):
    return jnp.maximum(x, 0.01 * x)


def _block_kernel(u_ref, i_ref, a_ref, o_ref, s_ref,
                  wui1a_ref, wui1b_ref, bui1_ref, wui2_ref, bui2_ref,
                  w1a_ref, w1b_ref, b1_ref, w2_ref, b2_ref, g_ref,
                  out_ref):
    f32 = jnp.float32
    h_ui = _leaky(jnp.dot(u_ref[...], wui1a_ref[...], preferred_element_type=f32)
                  + jnp.dot(i_ref[...], wui1b_ref[...], preferred_element_type=f32)
                  + bui1_ref[...])
    ui_emb = _leaky(jnp.dot(h_ui, wui2_ref[...], preferred_element_type=f32)
                    + bui2_ref[...])
    ui_t = jnp.concatenate([ui_emb] * R, axis=-1)

    h_all = _leaky(jnp.dot(a_ref[...], w1a_ref[...], preferred_element_type=f32)
                   + jnp.dot(o_ref[...], w1b_ref[...], preferred_element_type=f32)
                   + b1_ref[...])
    out_all = _leaky(jnp.dot(h_all, w2_ref[...], preferred_element_type=f32)
                     + b2_ref[...])

    ou = (out_all.reshape(BBLK, N, R * H) * ui_t[:, None, :]).reshape(TB, R * H)
    scores = jnp.dot(ou, g_ref[...],
                     preferred_element_type=f32).reshape(BBLK, N, R)
    oh = jax.lax.broadcasted_iota(jnp.int32, (BBLK, N, R), 2) == s_ref[...][:, :, None]
    out_ref[...] = jnp.sum(jnp.where(oh, scores, 0.0), axis=-1)


@jax.jit
def _run(u_emb, i_emb, a2, o2, s,
         wui1a, wui1b, bui1, Wui2, bui2, w1a, w1b, b1_all, w2_bd, b2_all, g):
    grid = (B // BBLK,)
    full = lambda shape: pl.BlockSpec(shape, lambda b: (0,) * len(shape))
    return pl.pallas_call(
        _block_kernel,
        grid=grid,
        in_specs=[
            pl.BlockSpec((BBLK, D), lambda b: (b, 0)),
            pl.BlockSpec((BBLK, D), lambda b: (b, 0)),
            pl.BlockSpec((TB, D), lambda b: (b, 0)),
            pl.BlockSpec((TB, D), lambda b: (b, 0)),
            pl.BlockSpec((BBLK, N), lambda b: (b, 0)),
            full((D, H)), full((D, H)), full((H,)), full((H, H)), full((H,)),
            full((D, R * H)), full((D, R * H)), full((R * H,)),
            full((R * H, R * H)), full((R * H,)), full((R * H, R)),
        ],
        out_specs=pl.BlockSpec((BBLK, N), lambda b: (b, 0)),
        out_shape=jax.ShapeDtypeStruct((B, N), jnp.float32),
        compiler_params=pltpu.CompilerParams(
            dimension_semantics=("arbitrary",),
        ),
    )(u_emb, i_emb, a2, o2, s,
      wui1a, wui1b, bui1, Wui2, bui2, w1a, w1b, b1_all, w2_bd, b2_all, g)


def kernel(u_emb, i_emb, a_emb, o_emb, s,
           Wui1, bui1, Wui2, bui2, Wao1, bao1, Wao2, bao2):
    w1_all = jnp.transpose(Wao1, (1, 0, 2)).reshape(2 * D, R * H)
    b1_all = bao1.reshape(R * H)
    eye = jnp.eye(R, dtype=Wao2.dtype)
    w2_bd = jnp.einsum('rkj,rq->rkqj', Wao2, eye).reshape(R * H, R * H)
    b2_all = bao2.reshape(R * H)
    g = jnp.repeat(jnp.eye(R, dtype=jnp.float32), H, axis=0)
    a2 = a_emb.reshape(B * N, D)
    o2 = o_emb.reshape(B * N, D)
    return _run(u_emb, i_emb, a2, o2, s,
                Wui1[:D], Wui1[D:], bui1, Wui2, bui2,
                w1_all[:D], w1_all[D:], b1_all, w2_bd, b2_all, g)


# speedup vs baseline: 1.6815x; 1.0469x over previous
import jax
import jax.numpy as jnp
from jax.experimental import pallas as pl
from jax.experimental.pallas import tpu as pltpu

B, N, D, H, R = 4096, 200, 16, 16, 8
BBLK = 64
TB = BBLK * N


def _leaky(x):
    return jnp.maximum(x, 0.01 * x)


def _block_kernel(u_ref, i_ref, a_ref, o_ref, s_ref,
                  wui1a_ref, wui1b_ref, bui1_ref, wui2_ref, bui2_ref,
                  w1a_ref, w1b_ref, b1_ref, w2_ref, b2_ref, g_ref,
                  out_ref):
    f32 = jnp.float32
    h_ui = _leaky(jnp.dot(u_ref[...], wui1a_ref[...], preferred_element_type=f32)
                  + jnp.dot(i_ref[...], wui1b_ref[...], preferred_element_type=f32)
                  + bui1_ref[...])
    ui_emb = _leaky(jnp.dot(h_ui, wui2_ref[...], preferred_element_type=f32)
                    + bui2_ref[...])
    ui_t = jnp.concatenate([ui_emb] * R, axis=-1)

    h_all = _leaky(jnp.dot(a_ref[...], w1a_ref[...], preferred_element_type=f32)
                   + jnp.dot(o_ref[...], w1b_ref[...], preferred_element_type=f32)
                   + b1_ref[...])
    out_all = _leaky(jnp.dot(h_all, w2_ref[...], preferred_element_type=f32)
                     + b2_ref[...])

    ou = (out_all.reshape(BBLK, N, R * H) * ui_t[:, None, :]).reshape(TB, R * H)
    scores = jnp.dot(ou, g_ref[...],
                     preferred_element_type=f32).reshape(BBLK, N, R)
    oh = jax.lax.broadcasted_iota(jnp.int32, (BBLK, N, R), 2) == s_ref[...][:, :, None]
    out_ref[...] = jnp.sum(jnp.where(oh, scores, 0.0), axis=-1)


@jax.jit
def _run(u_emb, i_emb, a2, o2, s,
         wui1a, wui1b, bui1, Wui2, bui2, w1a, w1b, b1_all, w2_bd, b2_all, g):
    grid = (B // BBLK,)
    full = lambda shape: pl.BlockSpec(shape, lambda b: (0,) * len(shape))
    return pl.pallas_call(
        _block_kernel,
        grid=grid,
        in_specs=[
            pl.BlockSpec((BBLK, D), lambda b: (b, 0)),
            pl.BlockSpec((BBLK, D), lambda b: (b, 0)),
            pl.BlockSpec((TB, D), lambda b: (b, 0)),
            pl.BlockSpec((TB, D), lambda b: (b, 0)),
            pl.BlockSpec((BBLK, N), lambda b: (b, 0)),
            full((D, H)), full((D, H)), full((H,)), full((H, H)), full((H,)),
            full((D, R * H)), full((D, R * H)), full((R * H,)),
            full((R * H, R * H)), full((R * H,)), full((R * H, R)),
        ],
        out_specs=pl.BlockSpec((BBLK, N), lambda b: (b, 0)),
        out_shape=jax.ShapeDtypeStruct((B, N), jnp.float32),
        compiler_params=pltpu.CompilerParams(
            dimension_semantics=("arbitrary",),
        ),
    )(u_emb, i_emb, a2, o2, s,
      wui1a, wui1b, bui1, Wui2, bui2, w1a, w1b, b1_all, w2_bd, b2_all, g)


def kernel(u_emb, i_emb, a_emb, o_emb, s,
           Wui1, bui1, Wui2, bui2, Wao1, bao1, Wao2, bao2):
    w1_all = jnp.transpose(Wao1, (1, 0, 2)).reshape(2 * D, R * H)
    b1_all = bao1.reshape(R * H)
    eye = jnp.eye(R, dtype=Wao2.dtype)
    w2_bd = jnp.einsum('rkj,rq->rkqj', Wao2, eye).reshape(R * H, R * H)
    b2_all = bao2.reshape(R * H)
    g = jnp.repeat(jnp.eye(R, dtype=jnp.float32), H, axis=0)
    a2 = a_emb.reshape(B * N, D)
    o2 = o_emb.reshape(B * N, D)
    return _run(u_emb, i_emb, a2, o2, s,
                Wui1[:D], Wui1[D:], bui1, Wui2, bui2,
                w1_all[:D], w1_all[D:], b1_all, w2_bd, b2_all, g)


# speedup vs baseline: 1.7036x; 1.0131x over previous
import jax
import jax.numpy as jnp
from jax.experimental import pallas as pl
from jax.experimental.pallas import tpu as pltpu

B, N, D, H, R = 4096, 200, 16, 16, 8
BBLK = 64
TB = BBLK * N
GRID = B // BBLK


def _leaky(x):
    return jnp.maximum(x, 0.01 * x)


def _block_kernel(u_ref, i_ref, a_ref, o_ref, s_ref,
                  wui1a_ref, wui1b_ref, bui1_ref, wui2_ref, bui2_ref,
                  w1a_ref, w1b_ref, b1_ref, w2_ref, b2_ref, gt_ref,
                  out_ref):
    f32 = jnp.float32
    h_ui = _leaky(jnp.dot(u_ref[...], wui1a_ref[...], preferred_element_type=f32)
                  + jnp.dot(i_ref[...], wui1b_ref[...], preferred_element_type=f32)
                  + bui1_ref[...])
    ui_emb = _leaky(jnp.dot(h_ui, wui2_ref[...], preferred_element_type=f32)
                    + bui2_ref[...])
    ui_t = jnp.concatenate([ui_emb] * R, axis=-1)

    h_all = _leaky(jnp.dot(a_ref[...], w1a_ref[...], preferred_element_type=f32)
                   + jnp.dot(o_ref[...], w1b_ref[...], preferred_element_type=f32)
                   + b1_ref[...])
    out_all = _leaky(jnp.dot(h_all, w2_ref[...], preferred_element_type=f32)
                     + b2_ref[...])

    ui_b = jnp.broadcast_to(ui_t[:, None, :], (BBLK, N, R * H)).reshape(TB, R * H)
    ou = out_all * ui_b

    scores_t = jax.lax.dot_general(
        gt_ref[...], ou, (((1,), (1,)), ((), ())),
        preferred_element_type=f32)

    s_row = s_ref[0]
    oh = jax.lax.broadcasted_iota(jnp.int32, (R, TB), 0) == s_row
    out_ref[...] = jnp.sum(jnp.where(oh, scores_t, 0.0), axis=0,
                           keepdims=True)[None]


@jax.jit
def _run(u_emb, i_emb, a2, o2, s3,
         wui1a, wui1b, bui1, Wui2, bui2, w1a, w1b, b1_all, w2_bd, b2_all, gt):
    full = lambda shape: pl.BlockSpec(shape, lambda b: (0,) * len(shape))
    out3 = pl.pallas_call(
        _block_kernel,
        grid=(GRID,),
        in_specs=[
            pl.BlockSpec((BBLK, D), lambda b: (b, 0)),
            pl.BlockSpec((BBLK, D), lambda b: (b, 0)),
            pl.BlockSpec((TB, D), lambda b: (b, 0)),
            pl.BlockSpec((TB, D), lambda b: (b, 0)),
            pl.BlockSpec((1, 1, TB), lambda b: (b, 0, 0)),
            full((D, H)), full((D, H)), full((H,)), full((H, H)), full((H,)),
            full((D, R * H)), full((D, R * H)), full((R * H,)),
            full((R * H, R * H)), full((R * H,)), full((R, R * H)),
        ],
        out_specs=pl.BlockSpec((1, 1, TB), lambda b: (b, 0, 0)),
        out_shape=jax.ShapeDtypeStruct((GRID, 1, TB), jnp.float32),
        compiler_params=pltpu.CompilerParams(
            dimension_semantics=("arbitrary",),
        ),
    )(u_emb, i_emb, a2, o2, s3,
      wui1a, wui1b, bui1, Wui2, bui2, w1a, w1b, b1_all, w2_bd, b2_all, gt)
    return out3.reshape(B, N)


def kernel(u_emb, i_emb, a_emb, o_emb, s,
           Wui1, bui1, Wui2, bui2, Wao1, bao1, Wao2, bao2):
    w1_all = jnp.transpose(Wao1, (1, 0, 2)).reshape(2 * D, R * H)
    b1_all = bao1.reshape(R * H)
    eye = jnp.eye(R, dtype=Wao2.dtype)
    w2_bd = jnp.einsum('rkj,rq->rkqj', Wao2, eye).reshape(R * H, R * H)
    b2_all = bao2.reshape(R * H)
    gt = jnp.repeat(jnp.eye(R, dtype=jnp.float32), H, axis=0).T
    a2 = a_emb.reshape(B * N, D)
    o2 = o_emb.reshape(B * N, D)
    s3 = s.reshape(GRID, 1, TB)
    return _run(u_emb, i_emb, a2, o2, s3,
                Wui1[:D], Wui1[D:], bui1, Wui2, bui2,
                w1_all[:D], w1_all[D:], b1_all, w2_bd, b2_all, gt)
